# baseline (speedup 1.0000x reference)
import functools

import jax
import jax.numpy as jnp
from jax import lax
from jax.experimental import pallas as pl
from jax.experimental.pallas import tpu as pltpu
from jax.experimental.pallas import tpu_sc as plsc

_DIM = 64
_B = 16384
_MARGIN = 1.0
_NC = 2
_NS = 16
_NW = _NC * _NS
_PER_W = _B // _NW
_CHUNK = 64
_NCHUNK = _PER_W // _CHUNK
_IDXS = (_NCHUNK, _CHUNK)


def _sc_body(ent, rel, *args):
    idx_hbm = args[:12]
    out = args[12]
    ivs = args[13:25]
    hb = args[25:27]
    rb = args[27:29]
    tb = args[29:31]
    pos_s = args[31]
    outv = args[32]
    sems = args[33:39]

    wid = lax.axis_index("s") * _NC + lax.axis_index("c")
    lanes = lax.iota(jnp.int32, 16)

    for iv, src in zip(ivs, idx_hbm):
        pltpu.sync_copy(src.at[wid], iv)

    def fire(c):
        side, j = divmod(c, _NCHUNK)
        s = c % 2
        ch = pltpu.async_copy(ent.at[ivs[6 * side + 0].at[j]], hb[s], sems[s * 3 + 0])
        cr = pltpu.async_copy(rel.at[ivs[6 * side + 2].at[j]], rb[s], sems[s * 3 + 1])
        ct = pltpu.async_copy(ent.at[ivs[6 * side + 4].at[j]], tb[s], sems[s * 3 + 2])
        return (ch, cr, ct)

    def compute(c, lacc):
        side, j = divmod(c, _NCHUNK)
        s = c % 2
        cbh_ref = ivs[6 * side + 1]
        cbr_ref = ivs[6 * side + 3]
        cbt_ref = ivs[6 * side + 5]

        def g_body(g, acc_loss):
            rows = lanes + g * 16
            cbh = cbh_ref[j, pl.ds(g * 16, 16)]
            cbr = cbr_ref[j, pl.ds(g * 16, 16)]
            cbt = cbt_ref[j, pl.ds(g * 16, 16)]
            acc = jnp.zeros((16,), jnp.float32)

            def d_body(dq, acc):
                d0 = dq * 16
                for dd in range(16):
                    offv = (lanes + (d0 + dd)) & 63
                    hv = plsc.load_gather(hb[s], [rows, cbh + offv])
                    rv = plsc.load_gather(rb[s], [rows, cbr + offv])
                    tv = plsc.load_gather(tb[s], [rows, cbt + offv])
                    acc = acc + jnp.abs(hv + rv - tv)
                return acc

            acc = lax.fori_loop(0, _DIM // 16, d_body, acc)
            start = j * _CHUNK + g * 16
            if side == 0:
                pos_s[pl.ds(start, 16)] = acc
                return acc_loss
            pv = pos_s[pl.ds(start, 16)]
            return acc_loss + jnp.maximum(pv - acc + _MARGIN, 0.0)

        return lax.fori_loop(0, _CHUNK // 16, g_body, lacc)

    lacc = jnp.zeros((16,), jnp.float32)
    pending = fire(0)
    for c in range(2 * _NCHUNK):
        nxt = fire(c + 1) if c + 1 < 2 * _NCHUNK else None
        for cp in pending:
            cp.wait()
        lacc = compute(c, lacc)
        pending = nxt

    outv[...] = lacc
    pltpu.sync_copy(outv, out.at[wid])


_sc_kernel = functools.partial(
    pl.kernel,
    out_type=jax.ShapeDtypeStruct((_NW, 16), jnp.float32),
    mesh=plsc.VectorSubcoreMesh(
        core_axis_name="c", subcore_axis_name="s",
        num_cores=_NC, num_subcores=_NS),
    compiler_params=pltpu.CompilerParams(
        needs_layout_passes=False, use_tc_tiling_on_sc=True),
    scratch_types=(
        [pltpu.VMEM(_IDXS, jnp.int32) for _ in range(12)]
        + [pltpu.VMEM((_CHUNK, 128), jnp.float32) for _ in range(6)]
        + [pltpu.VMEM((_PER_W,), jnp.float32),
           pltpu.VMEM((16,), jnp.float32)]
        + [pltpu.SemaphoreType.DMA for _ in range(6)]
    ),
)(_sc_body)


def _tr_body(a_ref, b_ref, o_ref):
    eye = jax.lax.broadcasted_iota(jnp.int32, (64, 128), 0)
    col = jax.lax.broadcasted_iota(jnp.int32, (64, 128), 1)
    ileft = (eye == col).astype(jnp.bfloat16)
    iright = (eye == col - 64).astype(jnp.bfloat16)
    dn = (((0,), (0,)), ((), ()))
    o_ref[...] = (
        jax.lax.dot_general(a_ref[...].astype(jnp.bfloat16), ileft, dn,
                            preferred_element_type=jnp.float32)
        + jax.lax.dot_general(b_ref[...].astype(jnp.bfloat16), iright, dn,
                              preferred_element_type=jnp.float32)
    )


def _transpose_to_pairs(x_t, split, blk):
    nb = split // blk
    last = (x_t.shape[1] + blk - 1) // blk - 1
    return pl.pallas_call(
        _tr_body,
        grid=(nb,),
        in_specs=[pl.BlockSpec((64, blk), lambda g: (0, g)),
                  pl.BlockSpec((64, blk),
                               lambda g, nb=nb, last=last:
                               (0, jnp.minimum(g + nb, last)))],
        out_specs=pl.BlockSpec((blk, 128), lambda g: (g, 0)),
        out_shape=jax.ShapeDtypeStruct((split, 128), jnp.float32),
    )(x_t, x_t)


def _tc_reduce_body(x_ref, o_ref):
    o_ref[...] = jnp.full((1, 1), jnp.sum(x_ref[...]) * (1.0 / _B), jnp.float32)


_tc_reduce = pl.pallas_call(
    _tc_reduce_body,
    out_shape=jax.ShapeDtypeStruct((1, 1), jnp.float32),
)


def kernel(entity_emb, relation_emb, h_pos, r_pos, t_pos, h_neg, r_neg, t_neg):
    ent2 = _transpose_to_pairs(entity_emb.T, 512000, 25600)
    rel2 = _transpose_to_pairs(relation_emb.T, 512, 512)
    shp = (_NW,) + _IDXS

    def split(i, sp):
        i = i.astype(jnp.int32)
        return jnp.where(i < sp, i, i - sp).reshape(shp), \
               jnp.where(i < sp, 0, 64).astype(jnp.int32).reshape(shp)

    hp, hpc = split(h_pos, 512000)
    rp, rpc = split(r_pos, 512)
    tp, tpc = split(t_pos, 512000)
    hn, hnc = split(h_neg, 512000)
    rn, rnc = split(r_neg, 512)
    tn, tnc = split(t_neg, 512000)
    parts = _sc_kernel(ent2, rel2,
                       hp, hpc, rp, rpc, tp, tpc,
                       hn, hnc, rn, rnc, tn, tnc)
    return _tc_reduce(parts).reshape(())


# speedup vs baseline: 1.0645x; 1.0645x over previous
import functools

import jax
import jax.numpy as jnp
from jax import lax
from jax.experimental import pallas as pl
from jax.experimental.pallas import tpu as pltpu
from jax.experimental.pallas import tpu_sc as plsc

_DIM = 64
_B = 16384
_MARGIN = 1.0
_NC = 2
_NS = 16
_NW = _NC * _NS
_PER_W = _B // _NW
_CHUNK = 128
_NCHUNK = _PER_W // _CHUNK
_IDXS = (_NCHUNK, _CHUNK)


def _sc_body(ent, rel, rows6, cbs6, *args):
    out = args[0]
    ivs = args[1:13]
    hb = args[13:15]
    rb = args[15:17]
    tb = args[17:19]
    pos_s = args[19]
    outv = args[20]
    sems = args[21:27]

    wid = lax.axis_index("s") * _NC + lax.axis_index("c")
    lanes = lax.iota(jnp.int32, 16)

    for k in range(6):
        pltpu.sync_copy(rows6.at[k, wid], ivs[2 * k])
        pltpu.sync_copy(cbs6.at[k, wid], ivs[2 * k + 1])

    def fire(c):
        side, j = divmod(c, _NCHUNK)
        s = c % 2
        ch = pltpu.async_copy(ent.at[ivs[6 * side + 0].at[j]], hb[s], sems[s * 3 + 0])
        cr = pltpu.async_copy(rel.at[ivs[6 * side + 2].at[j]], rb[s], sems[s * 3 + 1])
        ct = pltpu.async_copy(ent.at[ivs[6 * side + 4].at[j]], tb[s], sems[s * 3 + 2])

        return (ch, cr, ct)

    def compute(c, lacc):
        side, j = divmod(c, _NCHUNK)
        s = c % 2
        cbh_ref = ivs[6 * side + 1]
        cbr_ref = ivs[6 * side + 3]
        cbt_ref = ivs[6 * side + 5]

        def g_body(g, acc_loss):
            rows = lanes + g * 16
            cbh = cbh_ref[j, pl.ds(g * 16, 16)]
            cbr = cbr_ref[j, pl.ds(g * 16, 16)]
            cbt = cbt_ref[j, pl.ds(g * 16, 16)]
            acc = jnp.zeros((16,), jnp.float32)

            def d_body(dq, acc):
                d0 = dq * 16
                for dd in range(16):
                    offv = (lanes + (d0 + dd)) & 63
                    hv = plsc.load_gather(hb[s], [rows, cbh + offv])
                    rv = plsc.load_gather(rb[s], [rows, cbr + offv])
                    tv = plsc.load_gather(tb[s], [rows, cbt + offv])
                    acc = acc + jnp.abs(hv + rv - tv)
                return acc

            acc = lax.fori_loop(0, _DIM // 16, d_body, acc)
            start = j * _CHUNK + g * 16
            if side == 0:
                pos_s[pl.ds(start, 16)] = acc
                return acc_loss
            pv = pos_s[pl.ds(start, 16)]
            return acc_loss + jnp.maximum(pv - acc + _MARGIN, 0.0)

        return lax.fori_loop(0, _CHUNK // 16, g_body, lacc)

    lacc = jnp.zeros((16,), jnp.float32)
    pending = fire(0)
    for c in range(2 * _NCHUNK):
        nxt = fire(c + 1) if c + 1 < 2 * _NCHUNK else None
        for cp in pending:
            cp.wait()
        lacc = compute(c, lacc)
        pending = nxt

    outv[...] = lacc
    pltpu.sync_copy(outv, out.at[wid])


_sc_kernel = functools.partial(
    pl.kernel,
    out_type=jax.ShapeDtypeStruct((_NW, 16), jnp.float32),
    mesh=plsc.VectorSubcoreMesh(
        core_axis_name="c", subcore_axis_name="s",
        num_cores=_NC, num_subcores=_NS),
    compiler_params=pltpu.CompilerParams(
        needs_layout_passes=False, use_tc_tiling_on_sc=True),
    scratch_types=(
        [pltpu.VMEM(_IDXS, jnp.int32) for _ in range(12)]
        + [pltpu.VMEM((_CHUNK, 128), jnp.float32) for _ in range(6)]
        + [pltpu.VMEM((_PER_W,), jnp.float32),
           pltpu.VMEM((16,), jnp.float32)]
        + [pltpu.SemaphoreType.DMA for _ in range(6)]
    ),
)(_sc_body)


def _tr_body(a_ref, b_ref, o_ref):
    eye = jax.lax.broadcasted_iota(jnp.int32, (64, 128), 0)
    col = jax.lax.broadcasted_iota(jnp.int32, (64, 128), 1)
    ileft = (eye == col).astype(jnp.bfloat16)
    iright = (eye == col - 64).astype(jnp.bfloat16)
    dn = (((0,), (0,)), ((), ()))
    o_ref[...] = (
        jax.lax.dot_general(a_ref[...].astype(jnp.bfloat16), ileft, dn,
                            preferred_element_type=jnp.float32)
        + jax.lax.dot_general(b_ref[...].astype(jnp.bfloat16), iright, dn,
                              preferred_element_type=jnp.float32)
    )


def _transpose_to_pairs(x_t, split, blk):
    nb = split // blk
    last = (x_t.shape[1] + blk - 1) // blk - 1
    return pl.pallas_call(
        _tr_body,
        grid=(nb,),
        in_specs=[pl.BlockSpec((64, blk), lambda g: (0, g)),
                  pl.BlockSpec((64, blk),
                               lambda g, nb=nb, last=last:
                               (0, jnp.minimum(g + nb, last)))],
        out_specs=pl.BlockSpec((blk, 128), lambda g: (g, 0)),
        out_shape=jax.ShapeDtypeStruct((split, 128), jnp.float32),
    )(x_t, x_t)


def _tc_reduce_body(x_ref, o_ref):
    o_ref[...] = jnp.full((1, 1), jnp.sum(x_ref[...]) * (1.0 / _B), jnp.float32)


_tc_reduce = pl.pallas_call(
    _tc_reduce_body,
    out_shape=jax.ShapeDtypeStruct((1, 1), jnp.float32),
)


def kernel(entity_emb, relation_emb, h_pos, r_pos, t_pos, h_neg, r_neg, t_neg):
    ent2 = _transpose_to_pairs(entity_emb.T, 512000, 25600)
    rel2 = _transpose_to_pairs(relation_emb.T, 512, 512)
    shp = (6, _NW) + _IDXS

    raw = jnp.stack([h_pos, r_pos, t_pos, h_neg, r_neg, t_neg])
    sp = jnp.array([512000, 512, 512000, 512000, 512, 512000],
                   jnp.int32).reshape(6, 1)
    rows6 = jnp.where(raw < sp, raw, raw - sp).reshape(shp)
    cbs6 = jnp.where(raw < sp, 0, 64).astype(jnp.int32).reshape(shp)
    parts = _sc_kernel(ent2, rel2, rows6, cbs6)
    return _tc_reduce(parts).reshape(())
